# baseline (speedup 1.0000x reference)
import functools

import jax
import jax.numpy as jnp
import numpy as np
from jax import lax
from jax.experimental import pallas as pl
from jax.experimental.pallas import tpu as pltpu
from jax.experimental.pallas import tpu_sc as plsc

N = 10000
E = 320000
F = 128
ROW = 144
NPAD = 10240

NC = 2
NS = 16
NW = NC * NS
C = 64
CHUNKS = 157
EPW = C * CHUNKS
EPAD = NW * EPW
RPT = NPAD // NS


def _pre_body(x_ref, w1_ref, b1_ref, p_ref, q_ref):
    x = x_ref[...]
    w1a = w1_ref[:F, :]
    w1b = w1_ref[F:, :]
    q_ref[...] = jnp.dot(x, w1b,
                         preferred_element_type=jnp.float32).astype(jnp.bfloat16)
    p_ref[...] = (jnp.dot(x, w1a - w1b, preferred_element_type=jnp.float32)
                  + b1_ref[...]).astype(jnp.bfloat16)


def _pre(x, w1, b1):
    return pl.pallas_call(
        _pre_body,
        out_shape=(
            jax.ShapeDtypeStruct((NPAD, F), jnp.bfloat16),
            jax.ShapeDtypeStruct((NPAD, F), jnp.bfloat16),
        ),
    )(x, w1, b1)


def _edge_body(p_hbm, q_hbm, src_hbm, dst_hbm, zeros_hbm, out_hbm,
               idx0, idx1, idx2, idx3,
               prow0, qrow0, orow0, prow1, qrow1, orow1, acc,
               sem_i0, sem_i1, sem_i2, sem_i3,
               sem_p0, sem_q0, sem_s0, sem_p1, sem_q1, sem_s1):
    c = lax.axis_index("c")
    s = lax.axis_index("s")
    wid = c * NS + s
    ebase = wid * EPW

    pltpu.sync_copy(zeros_hbm.at[pl.ds(s * RPT, RPT)],
                    acc.at[pl.ds(s * RPT, RPT)])

    lane = lax.iota(jnp.int32, 16)
    count_pat = jnp.where(lane == 0, 1.0, 0.0).astype(jnp.float32)

    @plsc.parallel_loop(0, C)
    def _(r):
        orow0[r, pl.ds(F, 16)] = count_pat
        orow1[r, pl.ds(F, 16)] = count_pat

    plsc.subcore_barrier()

    ibufs = ((idx0, sem_i0), (idx1, sem_i1), (idx2, sem_i2), (idx3, sem_i3))
    dbufs = ((prow0, qrow0, orow0, sem_p0, sem_q0, sem_s0),
             (prow1, qrow1, orow1, sem_p1, sem_q1, sem_s1))

    def issue_idx(i, ib):
        idx, sem = ibufs[ib]
        base = ebase + i * C
        pltpu.async_copy(src_hbm.at[pl.ds(base, C)], idx.at[0], sem)
        pltpu.async_copy(dst_hbm.at[pl.ds(base, C)], idx.at[1], sem)

    def wait_idx(i, ib):
        idx, sem = ibufs[ib]
        base = ebase + i * C
        pltpu.make_async_copy(src_hbm.at[pl.ds(base, C)], idx.at[0], sem).wait()
        pltpu.make_async_copy(dst_hbm.at[pl.ds(base, C)], idx.at[1], sem).wait()

    def issue_gather(ib, db):
        idx = ibufs[ib][0]
        prow, qrow, _, sem_p, sem_q, _ = dbufs[db]
        pltpu.async_copy(p_hbm.at[idx.at[1]], prow, sem_p)
        pltpu.async_copy(q_hbm.at[idx.at[0]], qrow, sem_q)

    def wait_gather(ib, db):
        idx = ibufs[ib][0]
        prow, qrow, _, sem_p, sem_q, _ = dbufs[db]
        pltpu.make_async_copy(p_hbm.at[idx.at[1]], prow, sem_p).wait()
        pltpu.make_async_copy(q_hbm.at[idx.at[0]], qrow, sem_q).wait()

    def compute(db):
        prow, qrow, orow = dbufs[db][0], dbufs[db][1], dbufs[db][2]

        @plsc.parallel_loop(0, C, unroll=4)
        def _(r):
            for k in range(F // 32):
                sl = pl.ds(k * 32, 32)
                pe, po = plsc.unpack(prow[r, sl],
                                     format=plsc.PackFormat.INTERLEAVED,
                                     preferred_element_type=jnp.float32)
                qe, qo = plsc.unpack(qrow[r, sl],
                                     format=plsc.PackFormat.INTERLEAVED,
                                     preferred_element_type=jnp.float32)
                orow[r, pl.ds(k * 32, 16)] = jnp.maximum(pe + qe, 0.0)
                orow[r, pl.ds(k * 32 + 16, 16)] = jnp.maximum(po + qo, 0.0)

    def issue_scatter(ib, db):
        idx = ibufs[ib][0]
        orow, sem_s = dbufs[db][2], dbufs[db][5]
        pltpu.async_copy(orow, acc.at[idx.at[1]], sem_s, add=True)

    def wait_scatter(ib, db):
        idx = ibufs[ib][0]
        orow, sem_s = dbufs[db][2], dbufs[db][5]
        pltpu.make_async_copy(orow, acc.at[idx.at[1]], sem_s).wait()

    issue_idx(0, 0)
    issue_idx(1, 1)
    issue_idx(2, 2)
    issue_idx(3, 3)
    wait_idx(0, 0)
    issue_gather(0, 0)
    wait_idx(1, 1)
    issue_gather(1, 1)
    wait_gather(0, 0)
    compute(0)
    issue_scatter(0, 0)
    wait_idx(2, 2)
    issue_gather(2, 0)
    wait_gather(1, 1)
    compute(1)
    wait_scatter(0, 0)
    issue_scatter(1, 1)
    issue_idx(4, 0)
    wait_idx(3, 3)
    issue_gather(3, 1)

    def generic(i, ib, db, do_idx, do_gather):
        wait_gather(ib, db)
        compute(db)
        wait_scatter((ib - 1) % 4, 1 - db)
        issue_scatter(ib, db)
        if do_idx:
            issue_idx(i + 3, (ib + 3) % 4)
        if do_gather:
            wait_idx(i + 2, (ib + 2) % 4)
            issue_gather((ib + 2) % 4, db)

    def quad(u, carry):
        i0 = 4 * u + 2
        generic(i0, 2, 0, True, True)
        generic(i0 + 1, 3, 1, True, True)
        generic(i0 + 2, 0, 0, True, True)
        generic(i0 + 3, 1, 1, True, True)
        return carry

    lax.fori_loop(0, 38, quad, 0)

    generic(154, 2, 0, False, True)
    generic(155, 3, 1, False, False)
    generic(156, 0, 0, False, False)
    wait_scatter(0, 0)

    plsc.subcore_barrier()

    pltpu.sync_copy(acc.at[pl.ds(s * RPT, RPT)],
                    out_hbm.at[c, pl.ds(s * RPT, RPT)])


_edge = pl.kernel(
    _edge_body,
    out_type=jax.ShapeDtypeStruct((NC, NPAD, ROW), jnp.float32),
    mesh=plsc.VectorSubcoreMesh(core_axis_name="c", subcore_axis_name="s"),
    compiler_params=pltpu.CompilerParams(use_tc_tiling_on_sc=False,
                                         needs_layout_passes=False),
    scratch_types=[
        pltpu.VMEM((2, C), jnp.int32),
        pltpu.VMEM((2, C), jnp.int32),
        pltpu.VMEM((2, C), jnp.int32),
        pltpu.VMEM((2, C), jnp.int32),
        pltpu.VMEM((C, F), jnp.bfloat16),
        pltpu.VMEM((C, F), jnp.bfloat16),
        pltpu.VMEM((C, ROW), jnp.float32),
        pltpu.VMEM((C, F), jnp.bfloat16),
        pltpu.VMEM((C, F), jnp.bfloat16),
        pltpu.VMEM((C, ROW), jnp.float32),
        pltpu.VMEM_SHARED((NPAD, ROW), jnp.float32),
        pltpu.SemaphoreType.DMA,
        pltpu.SemaphoreType.DMA,
        pltpu.SemaphoreType.DMA,
        pltpu.SemaphoreType.DMA,
        pltpu.SemaphoreType.DMA,
        pltpu.SemaphoreType.DMA,
        pltpu.SemaphoreType.DMA,
        pltpu.SemaphoreType.DMA,
        pltpu.SemaphoreType.DMA,
        pltpu.SemaphoreType.DMA,
    ],
)


def _post_body(acc_ref, w2_ref, b2_ref, w3_ref, b3_ref, w4_ref, b4_ref,
               out_ref):
    a = acc_ref[0, :N, :] + acc_ref[1, :N, :]
    h_sum = a[:, :F]
    cnt = jnp.sum(a[:, F:ROW], axis=1, keepdims=True)
    denom = jnp.maximum(cnt, 1.0)
    summed = jnp.dot(h_sum, w2_ref[...], preferred_element_type=jnp.float32)
    agg = (summed + cnt * b2_ref[...]) / denom
    agg = jnp.maximum(agg, 0.0)
    h = jnp.maximum(
        jnp.dot(agg, w3_ref[...], preferred_element_type=jnp.float32)
        + b3_ref[...], 0.0)
    out_ref[...] = (jnp.dot(h, w4_ref[...], preferred_element_type=jnp.float32)
                    + b4_ref[...])


def _post(acc, w2, b2, w3, b3, w4, b4):
    return pl.pallas_call(
        _post_body,
        out_shape=jax.ShapeDtypeStruct((N, F), jnp.float32),
    )(acc, w2, b2, w3, b3, w4, b4)


_COLMAP = np.concatenate(
    [np.concatenate([32 * g + 2 * np.arange(16),
                     32 * g + 2 * np.arange(16) + 1]) for g in range(F // 32)])


def kernel(x, edge_index, W1, b1, W2, b2, W3, b3, W4, b4):
    src = edge_index[0].astype(jnp.int32)
    dst = edge_index[1].astype(jnp.int32)
    W2 = W2[_COLMAP, :]
    npad_e = EPAD - E
    pad_rows = (N + jnp.arange(npad_e, dtype=jnp.int32) % (NPAD - N))
    src = jnp.concatenate([src, pad_rows])
    dst = jnp.concatenate([dst, pad_rows])
    xp = jnp.concatenate([x, jnp.zeros((NPAD - N, F), x.dtype)])
    p, q = _pre(xp, W1, b1.reshape(1, F))
    zeros = jnp.zeros((NPAD, ROW), dtype=jnp.float32)
    acc = _edge(p, q, src, dst, zeros)
    return _post(acc, W2, b2.reshape(1, F), W3, b3.reshape(1, F // 2),
                 W4, b4.reshape(1, F))
